# Initial kernel scaffold; baseline (speedup 1.0000x reference)
import jax
import jax.numpy as jnp
from jax.experimental import pallas as pl


def kernel(x, edge_index, W1, b1, W_mu, b_mu, W_lv, b_lv):
    raise NotImplementedError("write your pallas kernel here")


# speedup vs baseline: 16.2386x; 16.2386x over previous
import functools

import jax
import jax.numpy as jnp
from jax import lax
from jax.experimental import pallas as pl
from jax.experimental.pallas import tpu as pltpu
from jax.experimental.pallas import tpu_sc as plsc

N = 10000
D_IN = 128
H1 = 128
H2 = 64

NC = 2
NS = 16
NW = NC * NS
CHUNK = 128
KCH = 79
PER_W = KCH * CHUNK
E_PAD = NW * PER_W
N_PAD = 10240
ROWS_PER_TILE = N_PAD // NS
BLK = 1024

_mesh = plsc.VectorSubcoreMesh(core_axis_name="c", subcore_axis_name="s")


@functools.partial(
    pl.kernel,
    out_type=jax.ShapeDtypeStruct((NC, N_PAD), jnp.float32),
    mesh=_mesh,
    scratch_types=[
        pltpu.VMEM_SHARED((N_PAD,), jnp.float32),
        pltpu.VMEM((CHUNK,), jnp.int32),
        pltpu.VMEM((CHUNK,), jnp.float32),
    ],
)
def _sc_degree(dst_hbm, zeros_hbm, out_hbm, acc_sh, dbuf, ones_v):
    c = lax.axis_index("c")
    s = lax.axis_index("s")
    pltpu.sync_copy(zeros_hbm, acc_sh.at[pl.ds(s * ROWS_PER_TILE, ROWS_PER_TILE)])
    for i in range(CHUNK // 16):
        ones_v[pl.ds(i * 16, 16)] = jnp.ones((16,), jnp.float32)
    plsc.subcore_barrier()
    base = (c * NS + s) * PER_W

    def step(k, carry):
        off = pl.multiple_of(base + k * CHUNK, CHUNK)
        pltpu.sync_copy(dst_hbm.at[pl.ds(off, CHUNK)], dbuf)
        pltpu.sync_copy(ones_v, acc_sh.at[dbuf], add=True)
        return carry

    lax.fori_loop(0, KCH, step, 0)
    plsc.subcore_barrier()
    nb = pl.multiple_of(s * ROWS_PER_TILE, ROWS_PER_TILE)
    pltpu.sync_copy(acc_sh.at[pl.ds(nb, ROWS_PER_TILE)],
                    out_hbm.at[c, pl.ds(nb, ROWS_PER_TILE)])


@functools.partial(
    pl.kernel,
    out_type=jax.ShapeDtypeStruct((NC, N_PAD, D_IN), jnp.float32),
    mesh=_mesh,
    scratch_types=[
        pltpu.VMEM_SHARED((N_PAD, D_IN), jnp.float32),
        pltpu.VMEM((CHUNK,), jnp.int32),
        pltpu.VMEM((CHUNK,), jnp.int32),
        pltpu.VMEM((CHUNK,), jnp.int32),
        pltpu.VMEM((CHUNK,), jnp.int32),
        pltpu.VMEM((CHUNK, D_IN), jnp.float32),
        pltpu.VMEM((CHUNK, D_IN), jnp.float32),
        pltpu.SemaphoreType.DMA,
        pltpu.SemaphoreType.DMA,
    ],
)
def _sc_aggregate(src_hbm, dst_hbm, tab_hbm, zrows_hbm, out_hbm,
                  acc_sh, si0, si1, di0, di1, r0, r1, sem0, sem1):
    c = lax.axis_index("c")
    s = lax.axis_index("s")
    pltpu.sync_copy(zrows_hbm, acc_sh.at[pl.ds(s * ROWS_PER_TILE, ROWS_PER_TILE)])
    plsc.subcore_barrier()
    base = (c * NS + s) * PER_W
    sib = (si0, si1)
    dib = (di0, di1)
    rb = (r0, r1)
    sems = (sem0, sem1)

    off0 = pl.multiple_of(base, CHUNK)
    pltpu.sync_copy(src_hbm.at[pl.ds(off0, CHUNK)], si0)
    pltpu.sync_copy(dst_hbm.at[pl.ds(off0, CHUNK)], di0)
    pltpu.async_copy(tab_hbm.at[si0], r0, sem0)

    def pair(j, carry):
        for b in range(2):
            k = 2 * j + 1 + b
            buf = (1 + b) % 2
            prev = 1 - buf
            off = pl.multiple_of(base + k * CHUNK, CHUNK)
            pltpu.sync_copy(src_hbm.at[pl.ds(off, CHUNK)], sib[buf])
            pltpu.sync_copy(dst_hbm.at[pl.ds(off, CHUNK)], dib[buf])
            pltpu.async_copy(tab_hbm.at[sib[buf]], rb[buf], sems[buf])
            pltpu.make_async_copy(tab_hbm.at[sib[prev]], rb[prev], sems[prev]).wait()
            pltpu.sync_copy(rb[prev], acc_sh.at[dib[prev]], add=True)
        return carry

    lax.fori_loop(0, (KCH - 1) // 2, pair, 0)
    pltpu.make_async_copy(tab_hbm.at[si0], r0, sem0).wait()
    pltpu.sync_copy(r0, acc_sh.at[di0], add=True)
    plsc.subcore_barrier()
    nb = pl.multiple_of(s * ROWS_PER_TILE, ROWS_PER_TILE)
    pltpu.sync_copy(acc_sh.at[pl.ds(nb, ROWS_PER_TILE)],
                    out_hbm.at[c, pl.ds(nb, ROWS_PER_TILE)])


def _dinv_block(deg_ref, block_idx):
    d2 = deg_ref[...]
    degsum = d2[:, 0:1] + d2[:, 1:2] + 1.0
    row = lax.broadcasted_iota(jnp.int32, (BLK, 1), 0) + block_idx * BLK
    return jnp.where(row < N, lax.rsqrt(degsum), 0.0)


def _tc_first_body(deg_ref, x_ref, w_ref, o_ref):
    dinv = _dinv_block(deg_ref, pl.program_id(0))
    z = jnp.dot(x_ref[...], w_ref[...], preferred_element_type=jnp.float32)
    o_ref[...] = z * dinv


_tc_first = pl.pallas_call(
    _tc_first_body,
    grid=(N_PAD // BLK,),
    in_specs=[
        pl.BlockSpec((BLK, NC), lambda i: (i, 0)),
        pl.BlockSpec((BLK, D_IN), lambda i: (i, 0)),
        pl.BlockSpec((D_IN, H1), lambda i: (0, 0)),
    ],
    out_specs=pl.BlockSpec((BLK, H1), lambda i: (i, 0)),
    out_shape=jax.ShapeDtypeStruct((N_PAD, H1), jnp.float32),
)


def _tc_mid_body(deg_ref, p_ref, z_ref, b_ref, o_ref):
    dinv = _dinv_block(deg_ref, pl.program_id(0))
    sfull = p_ref[0] + p_ref[1] + z_ref[...]
    h = jnp.maximum(dinv * sfull + b_ref[...], 0.0)
    o_ref[...] = dinv * h


_tc_mid = pl.pallas_call(
    _tc_mid_body,
    grid=(N_PAD // BLK,),
    in_specs=[
        pl.BlockSpec((BLK, NC), lambda i: (i, 0)),
        pl.BlockSpec((NC, BLK, H1), lambda i: (0, i, 0)),
        pl.BlockSpec((BLK, H1), lambda i: (i, 0)),
        pl.BlockSpec((1, H1), lambda i: (0, 0)),
    ],
    out_specs=pl.BlockSpec((BLK, H1), lambda i: (i, 0)),
    out_shape=jax.ShapeDtypeStruct((N_PAD, H1), jnp.float32),
)


def _tc_final_body(deg_ref, q_ref, h_ref, wmu_ref, bmu_ref, wlv_ref, blv_ref,
                   mu_ref, lv_ref):
    dinv = _dinv_block(deg_ref, pl.program_id(0))
    g = dinv * (q_ref[0] + q_ref[1] + h_ref[...])
    mu_ref[...] = jnp.dot(g, wmu_ref[...], preferred_element_type=jnp.float32) + bmu_ref[...]
    lv_ref[...] = jnp.dot(g, wlv_ref[...], preferred_element_type=jnp.float32) + blv_ref[...]


_tc_final = pl.pallas_call(
    _tc_final_body,
    grid=(N_PAD // BLK,),
    in_specs=[
        pl.BlockSpec((BLK, NC), lambda i: (i, 0)),
        pl.BlockSpec((NC, BLK, H1), lambda i: (0, i, 0)),
        pl.BlockSpec((BLK, H1), lambda i: (i, 0)),
        pl.BlockSpec((H1, H2), lambda i: (0, 0)),
        pl.BlockSpec((1, H2), lambda i: (0, 0)),
        pl.BlockSpec((H1, H2), lambda i: (0, 0)),
        pl.BlockSpec((1, H2), lambda i: (0, 0)),
    ],
    out_specs=[
        pl.BlockSpec((BLK, H2), lambda i: (i, 0)),
        pl.BlockSpec((BLK, H2), lambda i: (i, 0)),
    ],
    out_shape=[
        jax.ShapeDtypeStruct((N_PAD, H2), jnp.float32),
        jax.ShapeDtypeStruct((N_PAD, H2), jnp.float32),
    ],
)


def kernel(x, edge_index, W1, b1, W_mu, b_mu, W_lv, b_lv):
    ei = edge_index.astype(jnp.int32)
    pad = jnp.full((E_PAD - ei.shape[1],), N, jnp.int32)
    src = jnp.concatenate([ei[0], pad])
    dst = jnp.concatenate([ei[1], pad])
    x_pad = jnp.concatenate([x, jnp.zeros((N_PAD - N, D_IN), x.dtype)])
    zrows = jnp.zeros((ROWS_PER_TILE, D_IN), jnp.float32)
    zdeg = jnp.zeros((ROWS_PER_TILE,), jnp.float32)

    deg2 = _sc_degree(dst, zdeg)
    degT = deg2.T
    zp = _tc_first(degT, x_pad, W1)
    P = _sc_aggregate(src, dst, zp, zrows)
    hp = _tc_mid(degT, P, zp, b1.reshape(1, H1))
    Q = _sc_aggregate(src, dst, hp, zrows)
    mu_pad, lv_pad = _tc_final(degT, Q, hp, W_mu, b_mu.reshape(1, H2),
                               W_lv, b_lv.reshape(1, H2))
    return mu_pad[:N], lv_pad[:N]


# speedup vs baseline: 18.4637x; 1.1370x over previous
import functools

import jax
import jax.numpy as jnp
from jax import lax
from jax.experimental import pallas as pl
from jax.experimental.pallas import tpu as pltpu
from jax.experimental.pallas import tpu_sc as plsc

N = 10000
D_IN = 128
H1 = 128
H2 = 64

NC = 2
NS = 16
NW = NC * NS
CHUNK = 128
KCH = 79
PER_W = KCH * CHUNK
E_PAD = NW * PER_W
N_PAD = 10240
ROWS_PER_TILE = N_PAD // NS
BLK = 1024

_mesh = plsc.VectorSubcoreMesh(core_axis_name="c", subcore_axis_name="s")


@functools.partial(
    pl.kernel,
    out_type=jax.ShapeDtypeStruct((NC, N_PAD), jnp.float32),
    mesh=_mesh,
    scratch_types=[
        pltpu.VMEM_SHARED((N_PAD,), jnp.float32),
        pltpu.VMEM((CHUNK,), jnp.int32),
        pltpu.VMEM((CHUNK,), jnp.float32),
    ],
)
def _sc_degree(dst_hbm, zeros_hbm, out_hbm, acc_sh, dbuf, ones_v):
    c = lax.axis_index("c")
    s = lax.axis_index("s")
    pltpu.sync_copy(zeros_hbm, acc_sh.at[pl.ds(s * ROWS_PER_TILE, ROWS_PER_TILE)])
    for i in range(CHUNK // 16):
        ones_v[pl.ds(i * 16, 16)] = jnp.ones((16,), jnp.float32)
    plsc.subcore_barrier()
    base = (c * NS + s) * PER_W

    def step(k, carry):
        off = pl.multiple_of(base + k * CHUNK, CHUNK)
        pltpu.sync_copy(dst_hbm.at[pl.ds(off, CHUNK)], dbuf)
        pltpu.sync_copy(ones_v, acc_sh.at[dbuf], add=True)
        return carry

    lax.fori_loop(0, KCH, step, 0)
    plsc.subcore_barrier()
    nb = pl.multiple_of(s * ROWS_PER_TILE, ROWS_PER_TILE)
    pltpu.sync_copy(acc_sh.at[pl.ds(nb, ROWS_PER_TILE)],
                    out_hbm.at[c, pl.ds(nb, ROWS_PER_TILE)])


K0 = 106
K1 = 52


@functools.partial(
    pl.kernel,
    out_type=jax.ShapeDtypeStruct((NC, N_PAD, D_IN), jnp.float32),
    mesh=_mesh,
    scratch_types=[
        pltpu.VMEM_SHARED((N_PAD, D_IN), jnp.float32),
        pltpu.VMEM((CHUNK,), jnp.int32),
        pltpu.VMEM((CHUNK,), jnp.int32),
        pltpu.VMEM((CHUNK,), jnp.int32),
        pltpu.VMEM((CHUNK,), jnp.int32),
        pltpu.VMEM((CHUNK, D_IN), jnp.float32),
        pltpu.VMEM((CHUNK, D_IN), jnp.float32),
        pltpu.SemaphoreType.DMA,
        pltpu.SemaphoreType.DMA,
    ],
)
def _sc_aggregate(src_hbm, dst_hbm, tab_hbm, zrows_hbm, out_hbm,
                  acc_sh, si0, si1, di0, di1, r0, r1, sem0, sem1):
    c = lax.axis_index("c")
    s = lax.axis_index("s")
    pltpu.sync_copy(zrows_hbm, acc_sh.at[pl.ds(s * ROWS_PER_TILE, ROWS_PER_TILE)])
    plsc.subcore_barrier()
    nk = jnp.where(c == 0, K0, K1)
    base = jnp.where(c == 0, s * K0, NS * K0 + s * K1) * CHUNK
    npairs = jnp.where(c == 0, (K0 - 2) // 2, (K1 - 2) // 2)
    sib = (si0, si1)
    dib = (di0, di1)
    rb = (r0, r1)
    sems = (sem0, sem1)

    off0 = pl.multiple_of(base, CHUNK)
    pltpu.sync_copy(src_hbm.at[pl.ds(off0, CHUNK)], si0)
    pltpu.sync_copy(dst_hbm.at[pl.ds(off0, CHUNK)], di0)
    pltpu.async_copy(tab_hbm.at[si0], r0, sem0)

    def pair(j, carry):
        for b in range(2):
            k = 2 * j + 1 + b
            buf = (1 + b) % 2
            prev = 1 - buf
            off = pl.multiple_of(base + k * CHUNK, CHUNK)
            pltpu.sync_copy(src_hbm.at[pl.ds(off, CHUNK)], sib[buf])
            pltpu.sync_copy(dst_hbm.at[pl.ds(off, CHUNK)], dib[buf])
            pltpu.async_copy(tab_hbm.at[sib[buf]], rb[buf], sems[buf])
            pltpu.make_async_copy(tab_hbm.at[sib[prev]], rb[prev], sems[prev]).wait()
            pltpu.sync_copy(rb[prev], acc_sh.at[dib[prev]], add=True)
        return carry

    lax.fori_loop(0, npairs, pair, 0)
    off_last = pl.multiple_of(base + (nk - 1) * CHUNK, CHUNK)
    pltpu.sync_copy(src_hbm.at[pl.ds(off_last, CHUNK)], si1)
    pltpu.sync_copy(dst_hbm.at[pl.ds(off_last, CHUNK)], di1)
    pltpu.async_copy(tab_hbm.at[si1], r1, sem1)
    pltpu.make_async_copy(tab_hbm.at[si0], r0, sem0).wait()
    pltpu.sync_copy(r0, acc_sh.at[di0], add=True)
    pltpu.make_async_copy(tab_hbm.at[si1], r1, sem1).wait()
    pltpu.sync_copy(r1, acc_sh.at[di1], add=True)
    plsc.subcore_barrier()
    nb = pl.multiple_of(s * ROWS_PER_TILE, ROWS_PER_TILE)
    pltpu.sync_copy(acc_sh.at[pl.ds(nb, ROWS_PER_TILE)],
                    out_hbm.at[c, pl.ds(nb, ROWS_PER_TILE)])


def _dinv_block(deg_ref, block_idx):
    d2 = deg_ref[...]
    degsum = d2[:, 0:1] + d2[:, 1:2] + 1.0
    row = lax.broadcasted_iota(jnp.int32, (BLK, 1), 0) + block_idx * BLK
    return jnp.where(row < N, lax.rsqrt(degsum), 0.0)


def _tc_first_body(deg_ref, x_ref, w_ref, o_ref):
    dinv = _dinv_block(deg_ref, pl.program_id(0))
    z = jnp.dot(x_ref[...], w_ref[...], preferred_element_type=jnp.float32)
    o_ref[...] = z * dinv


_tc_first = pl.pallas_call(
    _tc_first_body,
    grid=(N_PAD // BLK,),
    in_specs=[
        pl.BlockSpec((BLK, NC), lambda i: (i, 0)),
        pl.BlockSpec((BLK, D_IN), lambda i: (i, 0)),
        pl.BlockSpec((D_IN, H1), lambda i: (0, 0)),
    ],
    out_specs=pl.BlockSpec((BLK, H1), lambda i: (i, 0)),
    out_shape=jax.ShapeDtypeStruct((N_PAD, H1), jnp.float32),
)


def _tc_mid_body(deg_ref, p_ref, z_ref, b_ref, o_ref):
    dinv = _dinv_block(deg_ref, pl.program_id(0))
    sfull = p_ref[0] + p_ref[1] + z_ref[...]
    h = jnp.maximum(dinv * sfull + b_ref[...], 0.0)
    o_ref[...] = dinv * h


_tc_mid = pl.pallas_call(
    _tc_mid_body,
    grid=(N_PAD // BLK,),
    in_specs=[
        pl.BlockSpec((BLK, NC), lambda i: (i, 0)),
        pl.BlockSpec((NC, BLK, H1), lambda i: (0, i, 0)),
        pl.BlockSpec((BLK, H1), lambda i: (i, 0)),
        pl.BlockSpec((1, H1), lambda i: (0, 0)),
    ],
    out_specs=pl.BlockSpec((BLK, H1), lambda i: (i, 0)),
    out_shape=jax.ShapeDtypeStruct((N_PAD, H1), jnp.float32),
)


def _tc_final_body(deg_ref, q_ref, h_ref, wmu_ref, bmu_ref, wlv_ref, blv_ref,
                   mu_ref, lv_ref):
    dinv = _dinv_block(deg_ref, pl.program_id(0))
    g = dinv * (q_ref[0] + q_ref[1] + h_ref[...])
    mu_ref[...] = jnp.dot(g, wmu_ref[...], preferred_element_type=jnp.float32) + bmu_ref[...]
    lv_ref[...] = jnp.dot(g, wlv_ref[...], preferred_element_type=jnp.float32) + blv_ref[...]


_tc_final = pl.pallas_call(
    _tc_final_body,
    grid=(N_PAD // BLK,),
    in_specs=[
        pl.BlockSpec((BLK, NC), lambda i: (i, 0)),
        pl.BlockSpec((NC, BLK, H1), lambda i: (0, i, 0)),
        pl.BlockSpec((BLK, H1), lambda i: (i, 0)),
        pl.BlockSpec((H1, H2), lambda i: (0, 0)),
        pl.BlockSpec((1, H2), lambda i: (0, 0)),
        pl.BlockSpec((H1, H2), lambda i: (0, 0)),
        pl.BlockSpec((1, H2), lambda i: (0, 0)),
    ],
    out_specs=[
        pl.BlockSpec((BLK, H2), lambda i: (i, 0)),
        pl.BlockSpec((BLK, H2), lambda i: (i, 0)),
    ],
    out_shape=[
        jax.ShapeDtypeStruct((N_PAD, H2), jnp.float32),
        jax.ShapeDtypeStruct((N_PAD, H2), jnp.float32),
    ],
)


def kernel(x, edge_index, W1, b1, W_mu, b_mu, W_lv, b_lv):
    ei = edge_index.astype(jnp.int32)
    pad = jnp.full((E_PAD - ei.shape[1],), N, jnp.int32)
    src = jnp.concatenate([ei[0], pad])
    dst = jnp.concatenate([ei[1], pad])
    x_pad = jnp.concatenate([x, jnp.zeros((N_PAD - N, D_IN), x.dtype)])
    zrows = jnp.zeros((ROWS_PER_TILE, D_IN), jnp.float32)
    zdeg = jnp.zeros((ROWS_PER_TILE,), jnp.float32)

    deg2 = _sc_degree(dst, zdeg)
    degT = deg2.T
    zp = _tc_first(degT, x_pad, W1)
    P = _sc_aggregate(src, dst, zp, zrows)
    hp = _tc_mid(degT, P, zp, b1.reshape(1, H1))
    Q = _sc_aggregate(src, dst, hp, zrows)
    mu_pad, lv_pad = _tc_final(degT, Q, hp, W_mu, b_mu.reshape(1, H2),
                               W_lv, b_lv.reshape(1, H2))
    return mu_pad[:N], lv_pad[:N]


# speedup vs baseline: 19.9424x; 1.0801x over previous
import functools

import jax
import jax.numpy as jnp
from jax import lax
from jax.experimental import pallas as pl
from jax.experimental.pallas import tpu as pltpu
from jax.experimental.pallas import tpu_sc as plsc

N = 10000
D_IN = 128
H1 = 128
H2 = 64

NC = 2
NS = 16
NW = NC * NS
CHUNK = 128
KCH = 79
PER_W = KCH * CHUNK
E_PAD = NW * PER_W
N_PAD = 10240
ROWS_PER_TILE = N_PAD // NS
BLK = 1024

_mesh = plsc.VectorSubcoreMesh(core_axis_name="c", subcore_axis_name="s")


@functools.partial(
    pl.kernel,
    out_type=jax.ShapeDtypeStruct((NC, N_PAD), jnp.float32),
    mesh=_mesh,
    scratch_types=[
        pltpu.VMEM_SHARED((N_PAD,), jnp.float32),
        pltpu.VMEM((CHUNK,), jnp.int32),
        pltpu.VMEM((CHUNK,), jnp.int32),
        pltpu.VMEM((CHUNK,), jnp.float32),
        pltpu.SemaphoreType.DMA,
        pltpu.SemaphoreType.DMA,
    ],
)
def _sc_degree(dst_hbm, zeros_hbm, out_hbm, acc_sh, db0, db1, ones_v, dsem0, dsem1):
    c = lax.axis_index("c")
    s = lax.axis_index("s")
    pltpu.sync_copy(zeros_hbm, acc_sh.at[pl.ds(s * ROWS_PER_TILE, ROWS_PER_TILE)])
    for i in range(CHUNK // 16):
        ones_v[pl.ds(i * 16, 16)] = jnp.ones((16,), jnp.float32)
    plsc.subcore_barrier()
    base = (c * NS + s) * PER_W
    dbufs = (db0, db1)
    dsems = (dsem0, dsem1)
    pltpu.async_copy(dst_hbm.at[pl.ds(pl.multiple_of(base, CHUNK), CHUNK)],
                     db0, dsem0)

    def step(j, carry):
        for b in range(2):
            k = 2 * j + 1 + b
            buf = (1 + b) % 2
            prev = 1 - buf
            off = pl.multiple_of(base + k * CHUNK, CHUNK)
            pltpu.async_copy(dst_hbm.at[pl.ds(off, CHUNK)], dbufs[buf],
                             dsems[buf])
            pltpu.make_async_copy(dst_hbm.at[pl.ds(off, CHUNK)], dbufs[prev],
                                  dsems[prev]).wait()
            pltpu.sync_copy(ones_v, acc_sh.at[dbufs[prev]], add=True)
        return carry

    lax.fori_loop(0, (KCH - 1) // 2, step, 0)
    off = pl.multiple_of(base, CHUNK)
    pltpu.make_async_copy(dst_hbm.at[pl.ds(off, CHUNK)], db0, dsem0).wait()
    pltpu.sync_copy(ones_v, acc_sh.at[db0], add=True)
    plsc.subcore_barrier()
    nb = pl.multiple_of(s * ROWS_PER_TILE, ROWS_PER_TILE)
    pltpu.sync_copy(acc_sh.at[pl.ds(nb, ROWS_PER_TILE)],
                    out_hbm.at[c, pl.ds(nb, ROWS_PER_TILE)])


K0 = 126
K1 = 32


@functools.partial(
    pl.kernel,
    out_type=jax.ShapeDtypeStruct((NC, N_PAD, D_IN), jnp.float32),
    mesh=_mesh,
    scratch_types=[
        pltpu.VMEM_SHARED((N_PAD, D_IN), jnp.float32),
        pltpu.VMEM((CHUNK,), jnp.int32),
        pltpu.VMEM((CHUNK,), jnp.int32),
        pltpu.VMEM((CHUNK,), jnp.int32),
        pltpu.VMEM((CHUNK,), jnp.int32),
        pltpu.VMEM((CHUNK, D_IN), jnp.float32),
        pltpu.VMEM((CHUNK, D_IN), jnp.float32),
        pltpu.SemaphoreType.DMA,
        pltpu.SemaphoreType.DMA,
    ],
)
def _sc_aggregate(src_hbm, dst_hbm, tab_hbm, zrows_hbm, out_hbm,
                  acc_sh, si0, si1, di0, di1, r0, r1, sem0, sem1):
    c = lax.axis_index("c")
    s = lax.axis_index("s")
    pltpu.sync_copy(zrows_hbm, acc_sh.at[pl.ds(s * ROWS_PER_TILE, ROWS_PER_TILE)])
    plsc.subcore_barrier()
    nk = jnp.where(c == 0, K0, K1)
    base = jnp.where(c == 0, s * K0, NS * K0 + s * K1) * CHUNK
    npairs = jnp.where(c == 0, (K0 - 2) // 2, (K1 - 2) // 2)
    sib = (si0, si1)
    dib = (di0, di1)
    rb = (r0, r1)
    sems = (sem0, sem1)

    off0 = pl.multiple_of(base, CHUNK)
    pltpu.sync_copy(src_hbm.at[pl.ds(off0, CHUNK)], si0)
    pltpu.sync_copy(dst_hbm.at[pl.ds(off0, CHUNK)], di0)
    pltpu.async_copy(tab_hbm.at[si0], r0, sem0)

    def pair(j, carry):
        for b in range(2):
            k = 2 * j + 1 + b
            buf = (1 + b) % 2
            prev = 1 - buf
            off = pl.multiple_of(base + k * CHUNK, CHUNK)
            pltpu.sync_copy(src_hbm.at[pl.ds(off, CHUNK)], sib[buf])
            pltpu.sync_copy(dst_hbm.at[pl.ds(off, CHUNK)], dib[buf])
            pltpu.async_copy(tab_hbm.at[sib[buf]], rb[buf], sems[buf])
            pltpu.make_async_copy(tab_hbm.at[sib[prev]], rb[prev], sems[prev]).wait()
            pltpu.sync_copy(rb[prev], acc_sh.at[dib[prev]], add=True)
        return carry

    lax.fori_loop(0, npairs, pair, 0)
    off_last = pl.multiple_of(base + (nk - 1) * CHUNK, CHUNK)
    pltpu.sync_copy(src_hbm.at[pl.ds(off_last, CHUNK)], si1)
    pltpu.sync_copy(dst_hbm.at[pl.ds(off_last, CHUNK)], di1)
    pltpu.async_copy(tab_hbm.at[si1], r1, sem1)
    pltpu.make_async_copy(tab_hbm.at[si0], r0, sem0).wait()
    pltpu.sync_copy(r0, acc_sh.at[di0], add=True)
    pltpu.make_async_copy(tab_hbm.at[si1], r1, sem1).wait()
    pltpu.sync_copy(r1, acc_sh.at[di1], add=True)
    plsc.subcore_barrier()
    nb = pl.multiple_of(s * ROWS_PER_TILE, ROWS_PER_TILE)
    pltpu.sync_copy(acc_sh.at[pl.ds(nb, ROWS_PER_TILE)],
                    out_hbm.at[c, pl.ds(nb, ROWS_PER_TILE)])


def _dinv_block(deg_ref, block_idx):
    d2 = deg_ref[...]
    degsum = d2[:, 0:1] + d2[:, 1:2] + 1.0
    row = lax.broadcasted_iota(jnp.int32, (BLK, 1), 0) + block_idx * BLK
    return jnp.where(row < N, lax.rsqrt(degsum), 0.0)


def _tc_first_body(deg_ref, x_ref, w_ref, o_ref):
    dinv = _dinv_block(deg_ref, pl.program_id(0))
    z = jnp.dot(x_ref[...], w_ref[...], preferred_element_type=jnp.float32)
    o_ref[...] = z * dinv


_tc_first = pl.pallas_call(
    _tc_first_body,
    grid=(N_PAD // BLK,),
    in_specs=[
        pl.BlockSpec((BLK, NC), lambda i: (i, 0)),
        pl.BlockSpec((BLK, D_IN), lambda i: (i, 0)),
        pl.BlockSpec((D_IN, H1), lambda i: (0, 0)),
    ],
    out_specs=pl.BlockSpec((BLK, H1), lambda i: (i, 0)),
    out_shape=jax.ShapeDtypeStruct((N_PAD, H1), jnp.float32),
)


def _tc_mid_body(deg_ref, p_ref, z_ref, b_ref, o_ref):
    dinv = _dinv_block(deg_ref, pl.program_id(0))
    sfull = p_ref[0] + p_ref[1] + z_ref[...]
    h = jnp.maximum(dinv * sfull + b_ref[...], 0.0)
    o_ref[...] = dinv * h


_tc_mid = pl.pallas_call(
    _tc_mid_body,
    grid=(N_PAD // BLK,),
    in_specs=[
        pl.BlockSpec((BLK, NC), lambda i: (i, 0)),
        pl.BlockSpec((NC, BLK, H1), lambda i: (0, i, 0)),
        pl.BlockSpec((BLK, H1), lambda i: (i, 0)),
        pl.BlockSpec((1, H1), lambda i: (0, 0)),
    ],
    out_specs=pl.BlockSpec((BLK, H1), lambda i: (i, 0)),
    out_shape=jax.ShapeDtypeStruct((N_PAD, H1), jnp.float32),
)


FBLK = 1000


def _dinv_final(deg_ref, block_idx):
    d2 = deg_ref[...]
    degsum = d2[:, 0:1] + d2[:, 1:2] + 1.0
    row = lax.broadcasted_iota(jnp.int32, (FBLK, 1), 0) + block_idx * FBLK
    return jnp.where(row < N, lax.rsqrt(degsum), 0.0)


def _tc_final_body(deg_ref, q_ref, h_ref, wmu_ref, bmu_ref, wlv_ref, blv_ref,
                   mu_ref, lv_ref):
    dinv = _dinv_final(deg_ref, pl.program_id(0))
    g = dinv * (q_ref[0] + q_ref[1] + h_ref[...])
    mu_ref[...] = jnp.dot(g, wmu_ref[...], preferred_element_type=jnp.float32) + bmu_ref[...]
    lv_ref[...] = jnp.dot(g, wlv_ref[...], preferred_element_type=jnp.float32) + blv_ref[...]


_tc_final = pl.pallas_call(
    _tc_final_body,
    grid=(N // FBLK,),
    in_specs=[
        pl.BlockSpec((FBLK, NC), lambda i: (i, 0)),
        pl.BlockSpec((NC, FBLK, H1), lambda i: (0, i, 0)),
        pl.BlockSpec((FBLK, H1), lambda i: (i, 0)),
        pl.BlockSpec((H1, H2), lambda i: (0, 0)),
        pl.BlockSpec((1, H2), lambda i: (0, 0)),
        pl.BlockSpec((H1, H2), lambda i: (0, 0)),
        pl.BlockSpec((1, H2), lambda i: (0, 0)),
    ],
    out_specs=[
        pl.BlockSpec((FBLK, H2), lambda i: (i, 0)),
        pl.BlockSpec((FBLK, H2), lambda i: (i, 0)),
    ],
    out_shape=[
        jax.ShapeDtypeStruct((N, H2), jnp.float32),
        jax.ShapeDtypeStruct((N, H2), jnp.float32),
    ],
)


def kernel(x, edge_index, W1, b1, W_mu, b_mu, W_lv, b_lv):
    ei = edge_index.astype(jnp.int32)
    pad = jnp.full((E_PAD - ei.shape[1],), N, jnp.int32)
    src = jnp.concatenate([ei[0], pad])
    dst = jnp.concatenate([ei[1], pad])
    x_pad = jnp.concatenate([x, jnp.zeros((N_PAD - N, D_IN), x.dtype)])
    zrows = jnp.zeros((ROWS_PER_TILE, D_IN), jnp.float32)
    zdeg = jnp.zeros((ROWS_PER_TILE,), jnp.float32)

    deg2 = _sc_degree(dst, zdeg)
    degT = deg2.T
    zp = _tc_first(degT, x_pad, W1)
    P = _sc_aggregate(src, dst, zp, zrows)
    hp = _tc_mid(degT, P, zp, b1.reshape(1, H1))
    Q = _sc_aggregate(src, dst, hp, zrows)
    mu, lv = _tc_final(degT, Q, hp, W_mu, b_mu.reshape(1, H2),
                       W_lv, b_lv.reshape(1, H2))
    return mu, lv


# speedup vs baseline: 20.3549x; 1.0207x over previous
import functools

import jax
import jax.numpy as jnp
from jax import lax
from jax.experimental import pallas as pl
from jax.experimental.pallas import tpu as pltpu
from jax.experimental.pallas import tpu_sc as plsc

N = 10000
D_IN = 128
H1 = 128
H2 = 64

NC = 2
NS = 16
NW = NC * NS
CHUNK = 128
KCH = 79
PER_W = KCH * CHUNK
E_PAD = NW * PER_W
N_PAD = 10240
ROWS_PER_TILE = N_PAD // NS
BLK = 1024
FBLK = 1000

_mesh = plsc.VectorSubcoreMesh(core_axis_name="c", subcore_axis_name="s")


@functools.partial(
    pl.kernel,
    out_type=jax.ShapeDtypeStruct((NC, N_PAD), jnp.float32),
    mesh=_mesh,
    scratch_types=[
        pltpu.VMEM_SHARED((N_PAD,), jnp.float32),
        pltpu.VMEM((CHUNK,), jnp.int32),
        pltpu.VMEM((CHUNK,), jnp.int32),
        pltpu.VMEM((CHUNK,), jnp.float32),
        pltpu.SemaphoreType.DMA,
        pltpu.SemaphoreType.DMA,
    ],
)
def _sc_degree(dst_hbm, zeros_hbm, out_hbm, acc_sh, db0, db1, ones_v, dsem0, dsem1):
    c = lax.axis_index("c")
    s = lax.axis_index("s")
    pltpu.sync_copy(zeros_hbm, acc_sh.at[pl.ds(s * ROWS_PER_TILE, ROWS_PER_TILE)])
    for i in range(CHUNK // 16):
        ones_v[pl.ds(i * 16, 16)] = jnp.ones((16,), jnp.float32)
    plsc.subcore_barrier()
    base = (c * NS + s) * PER_W
    dbufs = (db0, db1)
    dsems = (dsem0, dsem1)
    pltpu.async_copy(dst_hbm.at[pl.ds(pl.multiple_of(base, CHUNK), CHUNK)],
                     db0, dsem0)

    def step(j, carry):
        for b in range(2):
            k = 2 * j + 1 + b
            buf = (1 + b) % 2
            prev = 1 - buf
            off = pl.multiple_of(base + k * CHUNK, CHUNK)
            pltpu.async_copy(dst_hbm.at[pl.ds(off, CHUNK)], dbufs[buf],
                             dsems[buf])
            pltpu.make_async_copy(dst_hbm.at[pl.ds(off, CHUNK)], dbufs[prev],
                                  dsems[prev]).wait()
            pltpu.sync_copy(ones_v, acc_sh.at[dbufs[prev]], add=True)
        return carry

    lax.fori_loop(0, (KCH - 1) // 2, step, 0)
    off = pl.multiple_of(base, CHUNK)
    pltpu.make_async_copy(dst_hbm.at[pl.ds(off, CHUNK)], db0, dsem0).wait()
    pltpu.sync_copy(ones_v, acc_sh.at[db0], add=True)
    plsc.subcore_barrier()
    nb = pl.multiple_of(s * ROWS_PER_TILE, ROWS_PER_TILE)
    pltpu.sync_copy(acc_sh.at[pl.ds(nb, ROWS_PER_TILE)],
                    out_hbm.at[c, pl.ds(nb, ROWS_PER_TILE)])


K0 = 126
K1 = 32


@functools.partial(
    pl.kernel,
    out_type=jax.ShapeDtypeStruct((NC, N_PAD, D_IN), jnp.float32),
    mesh=_mesh,
    scratch_types=[
        pltpu.VMEM_SHARED((N_PAD, D_IN), jnp.float32),
        pltpu.VMEM((CHUNK,), jnp.int32),
        pltpu.VMEM((CHUNK,), jnp.int32),
        pltpu.VMEM((CHUNK,), jnp.int32),
        pltpu.VMEM((CHUNK,), jnp.int32),
        pltpu.VMEM((CHUNK, D_IN), jnp.float32),
        pltpu.VMEM((CHUNK, D_IN), jnp.float32),
        pltpu.SemaphoreType.DMA,
        pltpu.SemaphoreType.DMA,
    ],
)
def _sc_aggregate(src_hbm, dst_hbm, tab_hbm, zrows_hbm, out_hbm,
                  acc_sh, si0, si1, di0, di1, r0, r1, sem0, sem1):
    c = lax.axis_index("c")
    s = lax.axis_index("s")
    pltpu.sync_copy(zrows_hbm, acc_sh.at[pl.ds(s * ROWS_PER_TILE, ROWS_PER_TILE)])
    plsc.subcore_barrier()
    nk = jnp.where(c == 0, K0, K1)
    base = jnp.where(c == 0, s * K0, NS * K0 + s * K1) * CHUNK
    npairs = jnp.where(c == 0, (K0 - 2) // 2, (K1 - 2) // 2)
    sib = (si0, si1)
    dib = (di0, di1)
    rb = (r0, r1)
    sems = (sem0, sem1)

    off0 = pl.multiple_of(base, CHUNK)
    pltpu.sync_copy(src_hbm.at[pl.ds(off0, CHUNK)], si0)
    pltpu.sync_copy(dst_hbm.at[pl.ds(off0, CHUNK)], di0)
    pltpu.async_copy(tab_hbm.at[si0], r0, sem0)

    def pair(j, carry):
        for b in range(2):
            k = 2 * j + 1 + b
            buf = (1 + b) % 2
            prev = 1 - buf
            off = pl.multiple_of(base + k * CHUNK, CHUNK)
            pltpu.sync_copy(src_hbm.at[pl.ds(off, CHUNK)], sib[buf])
            pltpu.sync_copy(dst_hbm.at[pl.ds(off, CHUNK)], dib[buf])
            pltpu.async_copy(tab_hbm.at[sib[buf]], rb[buf], sems[buf])
            pltpu.make_async_copy(tab_hbm.at[sib[prev]], rb[prev], sems[prev]).wait()
            pltpu.sync_copy(rb[prev], acc_sh.at[dib[prev]], add=True)
        return carry

    lax.fori_loop(0, npairs, pair, 0)
    off_last = pl.multiple_of(base + (nk - 1) * CHUNK, CHUNK)
    pltpu.sync_copy(src_hbm.at[pl.ds(off_last, CHUNK)], si1)
    pltpu.sync_copy(dst_hbm.at[pl.ds(off_last, CHUNK)], di1)
    pltpu.async_copy(tab_hbm.at[si1], r1, sem1)
    pltpu.make_async_copy(tab_hbm.at[si0], r0, sem0).wait()
    pltpu.sync_copy(r0, acc_sh.at[di0], add=True)
    pltpu.make_async_copy(tab_hbm.at[si1], r1, sem1).wait()
    pltpu.sync_copy(r1, acc_sh.at[di1], add=True)
    plsc.subcore_barrier()
    nb = pl.multiple_of(s * ROWS_PER_TILE, ROWS_PER_TILE)
    pltpu.sync_copy(acc_sh.at[pl.ds(nb, ROWS_PER_TILE)],
                    out_hbm.at[c, pl.ds(nb, ROWS_PER_TILE)])


def _dinv_block(deg_ref, block_idx):
    d2 = deg_ref[...]
    degsum = d2[:, 0:1] + d2[:, 1:2] + 1.0
    row = lax.broadcasted_iota(jnp.int32, (BLK, 1), 0) + block_idx * BLK
    return jnp.where(row < N, lax.rsqrt(degsum), 0.0)


def _tc_first_body(deg_ref, x_ref, w_ref, o_ref):
    d2 = deg_ref[...]
    dinv = lax.rsqrt(d2[:, 0:1] + d2[:, 1:2] + 1.0)
    z = jnp.dot(x_ref[...], w_ref[...], preferred_element_type=jnp.float32)
    o_ref[...] = z * dinv


_tc_first = pl.pallas_call(
    _tc_first_body,
    grid=(N // FBLK,),
    in_specs=[
        pl.BlockSpec((FBLK, NC), lambda i: (i, 0)),
        pl.BlockSpec((FBLK, D_IN), lambda i: (i, 0)),
        pl.BlockSpec((D_IN, H1), lambda i: (0, 0)),
    ],
    out_specs=pl.BlockSpec((FBLK, H1), lambda i: (i, 0)),
    out_shape=jax.ShapeDtypeStruct((N_PAD, H1), jnp.float32),
)


def _tc_mid_body(deg_ref, p_ref, z_ref, b_ref, o_ref):
    dinv = _dinv_block(deg_ref, pl.program_id(0))
    sfull = p_ref[0] + p_ref[1] + z_ref[...]
    h = jnp.maximum(dinv * sfull + b_ref[...], 0.0)
    o_ref[...] = dinv * h


_tc_mid = pl.pallas_call(
    _tc_mid_body,
    grid=(N_PAD // BLK,),
    in_specs=[
        pl.BlockSpec((BLK, NC), lambda i: (i, 0)),
        pl.BlockSpec((NC, BLK, H1), lambda i: (0, i, 0)),
        pl.BlockSpec((BLK, H1), lambda i: (i, 0)),
        pl.BlockSpec((1, H1), lambda i: (0, 0)),
    ],
    out_specs=pl.BlockSpec((BLK, H1), lambda i: (i, 0)),
    out_shape=jax.ShapeDtypeStruct((N_PAD, H1), jnp.float32),
)


def _dinv_final(deg_ref, block_idx):
    d2 = deg_ref[...]
    degsum = d2[:, 0:1] + d2[:, 1:2] + 1.0
    row = lax.broadcasted_iota(jnp.int32, (FBLK, 1), 0) + block_idx * FBLK
    return jnp.where(row < N, lax.rsqrt(degsum), 0.0)


def _tc_final_body(deg_ref, q_ref, h_ref, wmu_ref, bmu_ref, wlv_ref, blv_ref,
                   mu_ref, lv_ref):
    dinv = _dinv_final(deg_ref, pl.program_id(0))
    g = dinv * (q_ref[0] + q_ref[1] + h_ref[...])
    mu_ref[...] = jnp.dot(g, wmu_ref[...], preferred_element_type=jnp.float32) + bmu_ref[...]
    lv_ref[...] = jnp.dot(g, wlv_ref[...], preferred_element_type=jnp.float32) + blv_ref[...]


_tc_final = pl.pallas_call(
    _tc_final_body,
    grid=(N // FBLK,),
    in_specs=[
        pl.BlockSpec((FBLK, NC), lambda i: (i, 0)),
        pl.BlockSpec((NC, FBLK, H1), lambda i: (0, i, 0)),
        pl.BlockSpec((FBLK, H1), lambda i: (i, 0)),
        pl.BlockSpec((H1, H2), lambda i: (0, 0)),
        pl.BlockSpec((1, H2), lambda i: (0, 0)),
        pl.BlockSpec((H1, H2), lambda i: (0, 0)),
        pl.BlockSpec((1, H2), lambda i: (0, 0)),
    ],
    out_specs=[
        pl.BlockSpec((FBLK, H2), lambda i: (i, 0)),
        pl.BlockSpec((FBLK, H2), lambda i: (i, 0)),
    ],
    out_shape=[
        jax.ShapeDtypeStruct((N, H2), jnp.float32),
        jax.ShapeDtypeStruct((N, H2), jnp.float32),
    ],
)


def kernel(x, edge_index, W1, b1, W_mu, b_mu, W_lv, b_lv):
    ei = edge_index.astype(jnp.int32)
    pad = jnp.full((E_PAD - ei.shape[1],), N, jnp.int32)
    src = jnp.concatenate([ei[0], pad])
    dst = jnp.concatenate([ei[1], pad])
    zrows = jnp.zeros((ROWS_PER_TILE, D_IN), jnp.float32)
    zdeg = jnp.zeros((ROWS_PER_TILE,), jnp.float32)

    deg2 = _sc_degree(dst, zdeg)
    degT = deg2.T
    zp = _tc_first(degT, x, W1)
    P = _sc_aggregate(src, dst, zp, zrows)
    hp = _tc_mid(degT, P, zp, b1.reshape(1, H1))
    Q = _sc_aggregate(src, dst, hp, zrows)
    mu, lv = _tc_final(degT, Q, hp, W_mu, b_mu.reshape(1, H2),
                       W_lv, b_lv.reshape(1, H2))
    return mu, lv
